# baseline (speedup 1.0000x reference)
import jax
import jax.numpy as jnp
from jax.experimental import pallas as pl
from jax.experimental.pallas import tpu as pltpu

E = 8
TOP_K = 2
N_GROUPS = 4
GS = E // N_GROUPS
TM = 128
TS = 256
NTBL = 64
HIGH = jax.lax.Precision.HIGHEST


def _routing(scores, T):
    lane = jax.lax.broadcasted_iota(jnp.int32, (T, E), 1)
    grp = lane // GS
    partner = jnp.where(lane % 2 == 0, jnp.roll(scores, -1, axis=1),
                        jnp.roll(scores, 1, axis=1))
    gsum = scores + partner
    g1 = jnp.argmax(gsum, axis=-1, keepdims=True) // GS
    gsum2 = jnp.where(grp == g1, -jnp.inf, gsum)
    g2 = jnp.argmax(gsum2, axis=-1, keepdims=True) // GS
    ms = jnp.where((grp == g1) | (grp == g2), scores, 0.0)
    i1 = jnp.argmax(ms, axis=-1, keepdims=True)
    v1 = jnp.max(ms, axis=-1, keepdims=True)
    ms2 = jnp.where(lane == i1, -jnp.inf, ms)
    i2 = jnp.argmax(ms2, axis=-1, keepdims=True)
    v2 = jnp.max(ms2, axis=-1, keepdims=True)
    den = v1 + v2 + 1e-20
    return i1, i2, v1 / den, v2 / den


def _dispatch_meta(scores, T, NPT):
    i1, i2, w1n, w2n = _routing(scores, T)
    lane8 = jax.lax.broadcasted_iota(jnp.int32, (T, E), 1)
    oh1 = (i1 == lane8).astype(jnp.float32)
    oh2 = (i2 == lane8).astype(jnp.float32)
    tri = (jax.lax.broadcasted_iota(jnp.int32, (T, T), 0)
           > jax.lax.broadcasted_iota(jnp.int32, (T, T), 1)).astype(jnp.float32)
    cum1 = jax.lax.dot_general(tri, oh1, (((1,), (0,)), ((), ())),
                               preferred_element_type=jnp.float32)
    cum2 = jax.lax.dot_general(tri, oh2, (((1,), (0,)), ((), ())),
                               preferred_element_type=jnp.float32)
    counts1 = jnp.sum(oh1, axis=0, keepdims=True)
    counts2 = jnp.sum(oh2, axis=0, keepdims=True)
    rank1 = jnp.sum(oh1 * cum1, axis=1, keepdims=True)
    rank2 = jnp.sum(oh2 * (cum2 + counts1), axis=1, keepdims=True)
    counts = counts1 + counts2
    pc = jnp.ceil(counts / TM) * TM
    ut = (jax.lax.broadcasted_iota(jnp.int32, (E, E), 0)
          <= jax.lax.broadcasted_iota(jnp.int32, (E, E), 1)).astype(jnp.float32)
    pc_cum = jax.lax.dot_general(pc, ut, (((1,), (0,)), ((), ())),
                                 preferred_element_type=jnp.float32)
    pcoff = (pc_cum - pc).astype(jnp.int32)
    pc_cum = pc_cum.astype(jnp.int32)
    g1r = jnp.sum(oh1 * pcoff.astype(jnp.float32), axis=1, keepdims=True)
    g2r = jnp.sum(oh2 * pcoff.astype(jnp.float32), axis=1, keepdims=True)
    row1 = rank1 + g1r
    row2 = rank2 + g2r
    meta_i = jnp.concatenate(
        [row1.astype(jnp.int32), row2.astype(jnp.int32),
         jnp.zeros((T, 6), jnp.int32)], axis=1)
    meta_f = jnp.concatenate([w1n, w2n, jnp.zeros((T, 6), jnp.float32)], axis=1)
    ti = jax.lax.broadcasted_iota(jnp.int32, (1, NTBL), 1) * TM
    texp = jnp.zeros((1, NTBL), jnp.int32)
    for e in range(E):
        texp += (ti >= pc_cum[0, e]).astype(jnp.int32)
    texp = jnp.minimum(texp, E - 1)
    tact = (ti < pc_cum[0, E - 1]).astype(jnp.int32)
    tbl = jnp.concatenate(
        [texp, tact, ti, jnp.zeros((5, NTBL), jnp.int32)], axis=0)
    return meta_i, meta_f, tbl


def _shared_routing_kernel(sc_ref, x_ref, ws1_ref, ws2_ref,
                           sh_ref, mi_ref, mf_ref, tbl_ref):
    t = pl.program_id(0)
    T = sc_ref.shape[0]

    @pl.when(t == 0)
    def _():
        mi, mf, tbl = _dispatch_meta(sc_ref[...], T, NTBL)
        mi_ref[...] = mi
        mf_ref[...] = mf
        tbl_ref[...] = tbl

    h = jax.nn.silu(jnp.dot(x_ref[...], ws1_ref[...],
                            preferred_element_type=jnp.float32))
    sh_ref[...] = jnp.dot(h, ws2_ref[...], preferred_element_type=jnp.float32)


def _ffn_kernel(tbl_ref, xf_ref, mi_ref, mf_ref, w1_ref, w2_ref, sh_ref,
                out_ref):
    p = pl.program_id(0)
    T = xf_ref.shape[0]

    @pl.when(p == 0)
    def _():
        out_ref[...] = sh_ref[...]

    e = tbl_ref[0, p]
    r0 = tbl_ref[2, p]

    @pl.when(tbl_ref[1, p] == 1)
    def _():
        row1 = mi_ref[:, 0:1]
        row2 = mi_ref[:, 1:2]
        lane_j = jax.lax.broadcasted_iota(jnp.int32, (T, TM), 1) + r0
        sel1 = row1 == lane_j
        sel2 = row2 == lane_j
        pt = (sel1 | sel2).astype(jnp.float32)
        xg = jax.lax.dot_general(pt, xf_ref[...], (((0,), (0,)), ((), ())),
                                 preferred_element_type=jnp.float32)
        h = jax.nn.silu(jnp.dot(xg, w1_ref[0],
                                preferred_element_type=jnp.float32))
        o = jnp.dot(h, w2_ref[0], preferred_element_type=jnp.float32)
        uw = jnp.where(sel1, mf_ref[:, 0:1], 0.0) + \
            jnp.where(sel2, mf_ref[:, 1:2], 0.0)
        out_ref[...] += jnp.dot(uw, o, preferred_element_type=jnp.float32)


def kernel(x, gate_w, w1, w2, ws1, ws2, bias):
    B, T, D = x.shape
    H = w1.shape[2]
    xf = x.reshape(T, D)
    scores = jax.nn.sigmoid(jnp.dot(xf, gate_w.T)) + bias[None, :]

    NTS = T // TS
    shared, meta_i, meta_f, tbl = pl.pallas_call(
        _shared_routing_kernel,
        grid=(NTS,),
        in_specs=[
            pl.BlockSpec((T, E), lambda t: (0, 0)),
            pl.BlockSpec((TS, D), lambda t: (t, 0)),
            pl.BlockSpec((D, H), lambda t: (0, 0)),
            pl.BlockSpec((H, D), lambda t: (0, 0)),
        ],
        out_specs=[
            pl.BlockSpec((TS, D), lambda t: (t, 0)),
            pl.BlockSpec((T, E), lambda t: (0, 0)),
            pl.BlockSpec((T, E), lambda t: (0, 0)),
            pl.BlockSpec((E, NTBL), lambda t: (0, 0)),
        ],
        out_shape=[
            jax.ShapeDtypeStruct((T, D), jnp.float32),
            jax.ShapeDtypeStruct((T, E), jnp.int32),
            jax.ShapeDtypeStruct((T, E), jnp.float32),
            jax.ShapeDtypeStruct((E, NTBL), jnp.int32),
        ],
        compiler_params=pltpu.CompilerParams(
            dimension_semantics=("arbitrary",),
        ),
    )(scores, xf, ws1, ws2)

    NPT = TOP_K * T // TM + E
    routed = pl.pallas_call(
        _ffn_kernel,
        grid_spec=pltpu.PrefetchScalarGridSpec(
            num_scalar_prefetch=1,
            grid=(NPT,),
            in_specs=[
                pl.BlockSpec((T, D), lambda p, tb: (0, 0)),
                pl.BlockSpec((T, E), lambda p, tb: (0, 0)),
                pl.BlockSpec((T, E), lambda p, tb: (0, 0)),
                pl.BlockSpec((1, D, H), lambda p, tb: (tb[0, p], 0, 0)),
                pl.BlockSpec((1, H, D), lambda p, tb: (tb[0, p], 0, 0)),
                pl.BlockSpec((T, D), lambda p, tb: (0, 0)),
            ],
            out_specs=pl.BlockSpec((T, D), lambda p, tb: (0, 0)),
        ),
        out_shape=jax.ShapeDtypeStruct((T, D), jnp.float32),
        compiler_params=pltpu.CompilerParams(
            dimension_semantics=("arbitrary",),
            vmem_limit_bytes=100 * 1024 * 1024,
        ),
    )(tbl, xf, meta_i, meta_f, w1, w2, shared)
    return routed.reshape(B, T, D)


# speedup vs baseline: 1.2840x; 1.2840x over previous
import jax
import jax.numpy as jnp
from jax.experimental import pallas as pl
from jax.experimental.pallas import tpu as pltpu

E = 8
TOP_K = 2
N_GROUPS = 4
GS = E // N_GROUPS
TM = 256
TS = 512
NTBL = 64
HIGH = jax.lax.Precision.HIGHEST


def _routing(scores, T):
    lane = jax.lax.broadcasted_iota(jnp.int32, (T, E), 1)
    grp = lane // GS
    partner = jnp.where(lane % 2 == 0, jnp.roll(scores, -1, axis=1),
                        jnp.roll(scores, 1, axis=1))
    gsum = scores + partner
    g1 = jnp.argmax(gsum, axis=-1, keepdims=True) // GS
    gsum2 = jnp.where(grp == g1, -jnp.inf, gsum)
    g2 = jnp.argmax(gsum2, axis=-1, keepdims=True) // GS
    ms = jnp.where((grp == g1) | (grp == g2), scores, 0.0)
    i1 = jnp.argmax(ms, axis=-1, keepdims=True)
    v1 = jnp.max(ms, axis=-1, keepdims=True)
    ms2 = jnp.where(lane == i1, -jnp.inf, ms)
    i2 = jnp.argmax(ms2, axis=-1, keepdims=True)
    v2 = jnp.max(ms2, axis=-1, keepdims=True)
    den = v1 + v2 + 1e-20
    return i1, i2, v1 / den, v2 / den


def _dispatch_meta(scores, T, NPT):
    i1, i2, w1n, w2n = _routing(scores, T)
    lane8 = jax.lax.broadcasted_iota(jnp.int32, (T, E), 1)
    oh1 = (i1 == lane8).astype(jnp.float32)
    oh2 = (i2 == lane8).astype(jnp.float32)
    tri = (jax.lax.broadcasted_iota(jnp.int32, (T, T), 0)
           > jax.lax.broadcasted_iota(jnp.int32, (T, T), 1)).astype(jnp.float32)
    cum1 = jax.lax.dot_general(tri, oh1, (((1,), (0,)), ((), ())),
                               preferred_element_type=jnp.float32)
    cum2 = jax.lax.dot_general(tri, oh2, (((1,), (0,)), ((), ())),
                               preferred_element_type=jnp.float32)
    counts1 = jnp.sum(oh1, axis=0, keepdims=True)
    counts2 = jnp.sum(oh2, axis=0, keepdims=True)
    rank1 = jnp.sum(oh1 * cum1, axis=1, keepdims=True)
    rank2 = jnp.sum(oh2 * (cum2 + counts1), axis=1, keepdims=True)
    counts = counts1 + counts2
    pc = jnp.ceil(counts / TM) * TM
    ut = (jax.lax.broadcasted_iota(jnp.int32, (E, E), 0)
          <= jax.lax.broadcasted_iota(jnp.int32, (E, E), 1)).astype(jnp.float32)
    pc_cum = jax.lax.dot_general(pc, ut, (((1,), (0,)), ((), ())),
                                 preferred_element_type=jnp.float32)
    pcoff = (pc_cum - pc).astype(jnp.int32)
    pc_cum = pc_cum.astype(jnp.int32)
    g1r = jnp.sum(oh1 * pcoff.astype(jnp.float32), axis=1, keepdims=True)
    g2r = jnp.sum(oh2 * pcoff.astype(jnp.float32), axis=1, keepdims=True)
    row1 = rank1 + g1r
    row2 = rank2 + g2r
    meta_i = jnp.concatenate(
        [row1.astype(jnp.int32), row2.astype(jnp.int32),
         jnp.zeros((T, 6), jnp.int32)], axis=1)
    meta_f = jnp.concatenate([w1n, w2n, jnp.zeros((T, 6), jnp.float32)], axis=1)
    ti = jax.lax.broadcasted_iota(jnp.int32, (1, NTBL), 1) * TM
    texp = jnp.zeros((1, NTBL), jnp.int32)
    for e in range(E):
        texp += (ti >= pc_cum[0, e]).astype(jnp.int32)
    texp = jnp.minimum(texp, E - 1)
    tact = (ti < pc_cum[0, E - 1]).astype(jnp.int32)
    tbl = jnp.concatenate(
        [texp, tact, ti, jnp.zeros((5, NTBL), jnp.int32)], axis=0)
    return meta_i, meta_f, tbl


def _shared_routing_kernel(sc_ref, x_ref, ws1_ref, ws2_ref,
                           sh_ref, mi_ref, mf_ref, tbl_ref):
    t = pl.program_id(0)
    T = sc_ref.shape[0]

    @pl.when(t == 0)
    def _():
        mi, mf, tbl = _dispatch_meta(sc_ref[...], T, NTBL)
        mi_ref[...] = mi
        mf_ref[...] = mf
        tbl_ref[...] = tbl

    h = jax.nn.silu(jnp.dot(x_ref[...], ws1_ref[...],
                            preferred_element_type=jnp.float32))
    sh_ref[...] = jnp.dot(h, ws2_ref[...], preferred_element_type=jnp.float32)


def _ffn_kernel(tbl_ref, xf_ref, mi_ref, mf_ref, w1_ref, w2_ref, sh_ref,
                out_ref):
    p = pl.program_id(0)
    T = xf_ref.shape[0]

    @pl.when(p == 0)
    def _():
        out_ref[...] = sh_ref[...]

    e = tbl_ref[0, p]
    r0 = tbl_ref[2, p]

    @pl.when(tbl_ref[1, p] == 1)
    def _():
        row1 = mi_ref[:, 0:1]
        row2 = mi_ref[:, 1:2]
        lane_j = jax.lax.broadcasted_iota(jnp.int32, (T, TM), 1) + r0
        sel1 = row1 == lane_j
        sel2 = row2 == lane_j
        pt = (sel1 | sel2).astype(jnp.float32)
        xg = jax.lax.dot_general(pt, xf_ref[...], (((0,), (0,)), ((), ())),
                                 preferred_element_type=jnp.float32)
        h = jax.nn.silu(jnp.dot(xg, w1_ref[0],
                                preferred_element_type=jnp.float32))
        o = jnp.dot(h, w2_ref[0], preferred_element_type=jnp.float32)
        uw = jnp.where(sel1, mf_ref[:, 0:1], 0.0) + \
            jnp.where(sel2, mf_ref[:, 1:2], 0.0)
        out_ref[...] += jnp.dot(uw, o, preferred_element_type=jnp.float32)


def kernel(x, gate_w, w1, w2, ws1, ws2, bias):
    B, T, D = x.shape
    H = w1.shape[2]
    xf = x.reshape(T, D)
    scores = jax.nn.sigmoid(jnp.dot(xf, gate_w.T)) + bias[None, :]

    NTS = T // TS
    shared, meta_i, meta_f, tbl = pl.pallas_call(
        _shared_routing_kernel,
        grid=(NTS,),
        in_specs=[
            pl.BlockSpec((T, E), lambda t: (0, 0)),
            pl.BlockSpec((TS, D), lambda t: (t, 0)),
            pl.BlockSpec((D, H), lambda t: (0, 0)),
            pl.BlockSpec((H, D), lambda t: (0, 0)),
        ],
        out_specs=[
            pl.BlockSpec((TS, D), lambda t: (t, 0)),
            pl.BlockSpec((T, E), lambda t: (0, 0)),
            pl.BlockSpec((T, E), lambda t: (0, 0)),
            pl.BlockSpec((E, NTBL), lambda t: (0, 0)),
        ],
        out_shape=[
            jax.ShapeDtypeStruct((T, D), jnp.float32),
            jax.ShapeDtypeStruct((T, E), jnp.int32),
            jax.ShapeDtypeStruct((T, E), jnp.float32),
            jax.ShapeDtypeStruct((E, NTBL), jnp.int32),
        ],
        compiler_params=pltpu.CompilerParams(
            dimension_semantics=("arbitrary",),
        ),
    )(scores, xf, ws1, ws2)

    NPT = TOP_K * T // TM + E
    routed = pl.pallas_call(
        _ffn_kernel,
        grid_spec=pltpu.PrefetchScalarGridSpec(
            num_scalar_prefetch=1,
            grid=(NPT,),
            in_specs=[
                pl.BlockSpec((T, D), lambda p, tb: (0, 0)),
                pl.BlockSpec((T, E), lambda p, tb: (0, 0)),
                pl.BlockSpec((T, E), lambda p, tb: (0, 0)),
                pl.BlockSpec((1, D, H), lambda p, tb: (tb[0, p], 0, 0)),
                pl.BlockSpec((1, H, D), lambda p, tb: (tb[0, p], 0, 0)),
                pl.BlockSpec((T, D), lambda p, tb: (0, 0)),
            ],
            out_specs=pl.BlockSpec((T, D), lambda p, tb: (0, 0)),
        ),
        out_shape=jax.ShapeDtypeStruct((T, D), jnp.float32),
        compiler_params=pltpu.CompilerParams(
            dimension_semantics=("arbitrary",),
            vmem_limit_bytes=100 * 1024 * 1024,
        ),
    )(tbl, xf, meta_i, meta_f, w1, w2, shared)
    return routed.reshape(B, T, D)


# speedup vs baseline: 1.2857x; 1.0014x over previous
import jax
import jax.numpy as jnp
from jax.experimental import pallas as pl
from jax.experimental.pallas import tpu as pltpu

E = 8
TOP_K = 2
N_GROUPS = 4
GS = E // N_GROUPS
TM = 256
TS = 512
NTBL = 64


def _routing(scores, T):
    lane = jax.lax.broadcasted_iota(jnp.int32, (T, E), 1)
    grp = lane // GS
    partner = jnp.where(lane % 2 == 0, jnp.roll(scores, -1, axis=1),
                        jnp.roll(scores, 1, axis=1))
    gsum = scores + partner
    g1 = jnp.argmax(gsum, axis=-1, keepdims=True) // GS
    gsum2 = jnp.where(grp == g1, -jnp.inf, gsum)
    g2 = jnp.argmax(gsum2, axis=-1, keepdims=True) // GS
    ms = jnp.where((grp == g1) | (grp == g2), scores, 0.0)
    i1 = jnp.argmax(ms, axis=-1, keepdims=True)
    v1 = jnp.max(ms, axis=-1, keepdims=True)
    ms2 = jnp.where(lane == i1, -jnp.inf, ms)
    i2 = jnp.argmax(ms2, axis=-1, keepdims=True)
    v2 = jnp.max(ms2, axis=-1, keepdims=True)
    den = v1 + v2 + 1e-20
    return i1, i2, v1 / den, v2 / den


def _dispatch_meta(scores, T, NPT):
    i1, i2, w1n, w2n = _routing(scores, T)
    lane8 = jax.lax.broadcasted_iota(jnp.int32, (T, E), 1)
    oh1 = (i1 == lane8).astype(jnp.float32)
    oh2 = (i2 == lane8).astype(jnp.float32)
    tri = (jax.lax.broadcasted_iota(jnp.int32, (T, T), 0)
           > jax.lax.broadcasted_iota(jnp.int32, (T, T), 1)).astype(jnp.float32)
    cum1 = jax.lax.dot_general(tri, oh1, (((1,), (0,)), ((), ())),
                               preferred_element_type=jnp.float32)
    cum2 = jax.lax.dot_general(tri, oh2, (((1,), (0,)), ((), ())),
                               preferred_element_type=jnp.float32)
    counts1 = jnp.sum(oh1, axis=0, keepdims=True)
    counts2 = jnp.sum(oh2, axis=0, keepdims=True)
    rank1 = jnp.sum(oh1 * cum1, axis=1, keepdims=True)
    rank2 = jnp.sum(oh2 * (cum2 + counts1), axis=1, keepdims=True)
    counts = counts1 + counts2
    pc = jnp.ceil(counts / TM) * TM
    ut = (jax.lax.broadcasted_iota(jnp.int32, (E, E), 0)
          <= jax.lax.broadcasted_iota(jnp.int32, (E, E), 1)).astype(jnp.float32)
    pc_cum = jax.lax.dot_general(pc, ut, (((1,), (0,)), ((), ())),
                                 preferred_element_type=jnp.float32)
    pcoff = (pc_cum - pc).astype(jnp.int32)
    pc_cum = pc_cum.astype(jnp.int32)
    g1r = jnp.sum(oh1 * pcoff.astype(jnp.float32), axis=1, keepdims=True)
    g2r = jnp.sum(oh2 * pcoff.astype(jnp.float32), axis=1, keepdims=True)
    row1 = rank1 + g1r
    row2 = rank2 + g2r
    meta_i = jnp.concatenate(
        [row1.astype(jnp.int32), row2.astype(jnp.int32),
         jnp.zeros((T, 6), jnp.int32)], axis=1)
    meta_f = jnp.concatenate([w1n, w2n, jnp.zeros((T, 6), jnp.float32)], axis=1)
    ti = jax.lax.broadcasted_iota(jnp.int32, (1, NTBL), 1) * TM
    texp = jnp.zeros((1, NTBL), jnp.int32)
    for e in range(E):
        texp += (ti >= pc_cum[0, e]).astype(jnp.int32)
    texp = jnp.minimum(texp, E - 1)
    tact = (ti < pc_cum[0, E - 1]).astype(jnp.int32)
    tbl = jnp.concatenate(
        [texp, tact, ti, jnp.zeros((5, NTBL), jnp.int32)], axis=0)
    return meta_i, meta_f, tbl


def _shared_routing_kernel(sc_ref, x_ref, ws1_ref, ws2_ref,
                           sh_ref, mi_ref, mf_ref, tbl_ref):
    t = pl.program_id(0)
    T = sc_ref.shape[0]

    @pl.when(t == 0)
    def _():
        mi, mf, tbl = _dispatch_meta(sc_ref[...], T, NTBL)
        mi_ref[...] = mi
        mf_ref[...] = mf
        tbl_ref[...] = tbl

    h = jax.nn.silu(jnp.dot(x_ref[...], ws1_ref[...],
                            preferred_element_type=jnp.float32))
    sh_ref[...] = jnp.dot(h, ws2_ref[...], preferred_element_type=jnp.float32)


def _ffn_kernel(tbl_ref, xf_ref, mi_ref, mf_ref, w1_ref, w2_ref, sh_ref,
                out_ref):
    p = pl.program_id(0)
    T = xf_ref.shape[0]

    @pl.when(p == 0)
    def _():
        out_ref[...] = sh_ref[...]

    r0 = tbl_ref[2, p]

    @pl.when(tbl_ref[1, p] == 1)
    def _():
        row1 = mi_ref[:, 0:1]
        row2 = mi_ref[:, 1:2]
        lane_j = jax.lax.broadcasted_iota(jnp.int32, (T, TM), 1) + r0
        sel1 = row1 == lane_j
        sel2 = row2 == lane_j
        pt = (sel1 | sel2).astype(jnp.float32)
        xg = jax.lax.dot_general(pt, xf_ref[...], (((0,), (0,)), ((), ())),
                                 preferred_element_type=jnp.float32)
        h = jax.nn.silu(jnp.dot(xg, w1_ref[0],
                                preferred_element_type=jnp.float32))
        o = jnp.dot(h, w2_ref[0], preferred_element_type=jnp.float32)
        uw = jnp.where(sel1, mf_ref[:, 0:1], 0.0) + \
            jnp.where(sel2, mf_ref[:, 1:2], 0.0)
        out_ref[...] += jnp.dot(uw, o, preferred_element_type=jnp.float32)


def kernel(x, gate_w, w1, w2, ws1, ws2, bias):
    B, T, D = x.shape
    H = w1.shape[2]
    xf = x.reshape(T, D)
    scores = jax.nn.sigmoid(jnp.dot(xf, gate_w.T)) + bias[None, :]

    NTS = T // TS
    shared, meta_i, meta_f, tbl = pl.pallas_call(
        _shared_routing_kernel,
        grid=(NTS,),
        in_specs=[
            pl.BlockSpec((T, E), lambda t: (0, 0)),
            pl.BlockSpec((TS, D), lambda t: (t, 0)),
            pl.BlockSpec((D, H), lambda t: (0, 0)),
            pl.BlockSpec((H, D), lambda t: (0, 0)),
        ],
        out_specs=[
            pl.BlockSpec((TS, D), lambda t: (t, 0)),
            pl.BlockSpec((T, E), lambda t: (0, 0)),
            pl.BlockSpec((T, E), lambda t: (0, 0)),
            pl.BlockSpec((E, NTBL), lambda t: (0, 0)),
        ],
        out_shape=[
            jax.ShapeDtypeStruct((T, D), jnp.float32),
            jax.ShapeDtypeStruct((T, E), jnp.int32),
            jax.ShapeDtypeStruct((T, E), jnp.float32),
            jax.ShapeDtypeStruct((E, NTBL), jnp.int32),
        ],
        compiler_params=pltpu.CompilerParams(
            dimension_semantics=("arbitrary",),
        ),
    )(scores, xf, ws1, ws2)

    NPT = TOP_K * T // TM + E
    routed = pl.pallas_call(
        _ffn_kernel,
        grid_spec=pltpu.PrefetchScalarGridSpec(
            num_scalar_prefetch=1,
            grid=(NPT,),
            in_specs=[
                pl.BlockSpec((T, D), lambda p, tb: (0, 0)),
                pl.BlockSpec((T, E), lambda p, tb: (0, 0)),
                pl.BlockSpec((T, E), lambda p, tb: (0, 0)),
                pl.BlockSpec((1, D, H), lambda p, tb: (tb[0, p], 0, 0)),
                pl.BlockSpec((1, H, D), lambda p, tb: (tb[0, p], 0, 0)),
                pl.BlockSpec((T, D), lambda p, tb: (0, 0)),
            ],
            out_specs=pl.BlockSpec((T, D), lambda p, tb: (0, 0)),
        ),
        out_shape=jax.ShapeDtypeStruct((T, D), jnp.float32),
        compiler_params=pltpu.CompilerParams(
            dimension_semantics=("arbitrary",),
            vmem_limit_bytes=100 * 1024 * 1024,
        ),
    )(tbl, xf, meta_i, meta_f, w1, w2, shared)
    return routed.reshape(B, T, D)
